# baseline (speedup 1.0000x reference)
import functools

import jax
import jax.numpy as jnp
from jax import lax
from jax.experimental import pallas as pl
from jax.experimental.pallas import tpu as pltpu
from jax.experimental.pallas import tpu_sc as plsc

_BATCHER8 = [
    (0, 1), (2, 3), (4, 5), (6, 7),
    (0, 2), (1, 3), (4, 6), (5, 7),
    (1, 2), (5, 6),
    (0, 4), (1, 5), (2, 6), (3, 7),
    (2, 4), (3, 5),
    (1, 2), (3, 4), (5, 6),
]

_BITONIC8 = [
    (0, 4), (1, 5), (2, 6), (3, 7),
    (0, 2), (1, 3), (4, 6), (5, 7),
    (0, 1), (2, 3), (4, 5), (6, 7),
]


def _ce(a, i, j):
    hi = jnp.maximum(a[i], a[j])
    lo = jnp.minimum(a[i], a[j])
    a[i] = hi
    a[j] = lo


def _merge_keep_top(a, b):
    m = [jnp.maximum(a[i], b[7 - i]) for i in range(8)]
    for (i, j) in _BITONIC8:
        _ce(m, i, j)
    return m


def _sorted_group(x_ref, b, start):
    g = [x_ref[b, pl.ds(pl.multiple_of(start + 8 * k, 8), 8), :]
         for k in range(8)]
    for (i, j) in _BATCHER8:
        _ce(g, i, j)
    return g


def _tc_body(x_ref, o_ref):
    NB, S, C = x_ref.shape
    niter = S // 128

    for b in range(NB):
        neg = jnp.full((8, C), -jnp.inf, jnp.float32)

        def step(m, carry):
            st0, st1 = carry
            base = m * 128
            st0 = _merge_keep_top(list(st0), _sorted_group(x_ref, b, base))
            st1 = _merge_keep_top(list(st1), _sorted_group(x_ref, b, base + 64))
            return (tuple(st0), tuple(st1))

        st0, st1 = lax.fori_loop(
            0, niter, step, (tuple([neg] * 8), tuple([neg] * 8)), unroll=4)
        c = _merge_keep_top(list(st0), list(st1))
        for shift in (4, 2, 1):
            rolled = [pltpu.roll(c[k], shift, axis=0) for k in range(8)]
            c = _merge_keep_top(c, rolled)
        for k in range(8):
            o_ref[b, k:k + 1, :] = c[k][0:1, :]


def _tc_topk(x):
    B, S, C = x.shape
    NB = 4
    return pl.pallas_call(
        _tc_body,
        grid=(B // NB,),
        in_specs=[pl.BlockSpec((NB, S, C), lambda b: (b, 0, 0))],
        out_specs=pl.BlockSpec((NB, 8, C), lambda b: (b, 0, 0)),
        out_shape=jax.ShapeDtypeStruct((B, 8, C), x.dtype),
    )(x)


_N_SC = 4
_SHARDS = 32
_L = 16


def _sc_topk_partials(x_sc):
    n_sc, S, C = x_sc.shape
    rows = S // _SHARDS
    ncv = C // _L

    @functools.partial(
        pl.kernel,
        mesh=plsc.VectorSubcoreMesh(core_axis_name="c", subcore_axis_name="s"),
        out_type=jax.ShapeDtypeStruct((n_sc, _SHARDS, 8, C), jnp.float32),
        scratch_types=[
            pltpu.VMEM((rows, C), jnp.float32),
            pltpu.VMEM((8, C), jnp.float32),
        ],
    )
    def sc_kernel(x_hbm, part_hbm, buf, pout):
        wid = lax.axis_index("s") * 2 + lax.axis_index("c")
        start = wid * rows
        for b in range(n_sc):
            pltpu.sync_copy(x_hbm.at[b, pl.ds(start, rows)], buf)
            for cv in range(ncv):
                neg = jnp.full((_L,), -jnp.inf, jnp.float32)

                def step(m, st, _cv=cv):
                    g = [buf[m * 8 + k, pl.ds(_cv * _L, _L)] for k in range(8)]
                    for (i, j) in _BATCHER8:
                        _ce(g, i, j)
                    return tuple(_merge_keep_top(list(st), g))

                st = lax.fori_loop(0, rows // 8, step, (neg,) * 8)
                for k in range(8):
                    pout[k, pl.ds(cv * _L, _L)] = st[k]
            pltpu.sync_copy(pout, part_hbm.at[b, wid])

    return sc_kernel(x_sc)


def _sc_merge_body(p_ref, o_ref):
    lists = [[p_ref[0, t, k:k + 1, :] for k in range(8)]
             for t in range(_SHARDS)]
    while len(lists) > 1:
        lists = [_merge_keep_top(lists[2 * i], lists[2 * i + 1])
                 for i in range(len(lists) // 2)]
    for k in range(8):
        o_ref[0, k:k + 1, :] = lists[0][k]


def _sc_merge(part):
    n_sc, nsh, _, C = part.shape
    return pl.pallas_call(
        _sc_merge_body,
        grid=(n_sc,),
        in_specs=[pl.BlockSpec((1, nsh, 8, C), lambda b: (b, 0, 0, 0))],
        out_specs=pl.BlockSpec((1, 8, C), lambda b: (b, 0, 0)),
        out_shape=jax.ShapeDtypeStruct((n_sc, 8, C), jnp.float32),
    )(part)


def kernel(x):
    B, S, C = x.shape
    n_tc = B - _N_SC
    out_tc = _tc_topk(x[:n_tc])
    part = _sc_topk_partials(x[n_tc:])
    out_sc = _sc_merge(part)
    out = jnp.concatenate([out_tc, out_sc], axis=0)
    return out.reshape(B, 8 * C)


# speedup vs baseline: 1.0055x; 1.0055x over previous
import functools

import jax
import jax.numpy as jnp
from jax import lax
from jax.experimental import pallas as pl
from jax.experimental.pallas import tpu as pltpu
from jax.experimental.pallas import tpu_sc as plsc

_BATCHER8 = [
    (0, 1), (2, 3), (4, 5), (6, 7),
    (0, 2), (1, 3), (4, 6), (5, 7),
    (1, 2), (5, 6),
    (0, 4), (1, 5), (2, 6), (3, 7),
    (2, 4), (3, 5),
    (1, 2), (3, 4), (5, 6),
]

_BITONIC8 = [
    (0, 4), (1, 5), (2, 6), (3, 7),
    (0, 2), (1, 3), (4, 6), (5, 7),
    (0, 1), (2, 3), (4, 5), (6, 7),
]


def _ce(a, i, j):
    hi = jnp.maximum(a[i], a[j])
    lo = jnp.minimum(a[i], a[j])
    a[i] = hi
    a[j] = lo


def _merge_keep_top(a, b):
    m = [jnp.maximum(a[i], b[7 - i]) for i in range(8)]
    for (i, j) in _BITONIC8:
        _ce(m, i, j)
    return m


def _sorted_group(x_ref, b, start):
    g = [x_ref[b, pl.ds(pl.multiple_of(start + 8 * k, 8), 8), :]
         for k in range(8)]
    for (i, j) in _BATCHER8:
        _ce(g, i, j)
    return g


def _tc_body(x_ref, o_ref):
    NB, S, C = x_ref.shape
    niter = S // 128

    for b in range(NB):
        neg = jnp.full((8, C), -jnp.inf, jnp.float32)

        def step(m, carry):
            st0, st1 = carry
            base = m * 128
            st0 = _merge_keep_top(list(st0), _sorted_group(x_ref, b, base))
            st1 = _merge_keep_top(list(st1), _sorted_group(x_ref, b, base + 64))
            return (tuple(st0), tuple(st1))

        st0, st1 = lax.fori_loop(
            0, niter, step, (tuple([neg] * 8), tuple([neg] * 8)), unroll=4)
        c = _merge_keep_top(list(st0), list(st1))
        for shift in (4, 2, 1):
            rolled = [pltpu.roll(c[k], shift, axis=0) for k in range(8)]
            c = _merge_keep_top(c, rolled)
        for k in range(8):
            o_ref[b, k:k + 1, :] = c[k][0:1, :]


def _tc_topk(x):
    B, S, C = x.shape
    NB = next(nb for nb in (5, 4, 3, 2, 1) if B % nb == 0)
    return pl.pallas_call(
        _tc_body,
        grid=(B // NB,),
        in_specs=[pl.BlockSpec((NB, S, C), lambda b: (b, 0, 0))],
        out_specs=pl.BlockSpec((NB, 8, C), lambda b: (b, 0, 0)),
        out_shape=jax.ShapeDtypeStruct((B, 8, C), x.dtype),
    )(x)


_N_SC = 2
_SHARDS = 32
_L = 16


def _sc_topk_partials(x_sc):
    n_sc, S, C = x_sc.shape
    rows = S // _SHARDS
    ncv = C // _L

    @functools.partial(
        pl.kernel,
        mesh=plsc.VectorSubcoreMesh(core_axis_name="c", subcore_axis_name="s"),
        out_type=jax.ShapeDtypeStruct((n_sc, _SHARDS, 8, C), jnp.float32),
        scratch_types=[
            pltpu.VMEM((rows, C), jnp.float32),
            pltpu.VMEM((8, C), jnp.float32),
        ],
    )
    def sc_kernel(x_hbm, part_hbm, buf, pout):
        wid = lax.axis_index("s") * 2 + lax.axis_index("c")
        start = wid * rows
        for b in range(n_sc):
            pltpu.sync_copy(x_hbm.at[b, pl.ds(start, rows)], buf)
            for cv in range(ncv):
                neg = jnp.full((_L,), -jnp.inf, jnp.float32)

                def step(m, st, _cv=cv):
                    g = [buf[m * 8 + k, pl.ds(_cv * _L, _L)] for k in range(8)]
                    for (i, j) in _BATCHER8:
                        _ce(g, i, j)
                    return tuple(_merge_keep_top(list(st), g))

                st = lax.fori_loop(0, rows // 8, step, (neg,) * 8)
                for k in range(8):
                    pout[k, pl.ds(cv * _L, _L)] = st[k]
            pltpu.sync_copy(pout, part_hbm.at[b, wid])

    return sc_kernel(x_sc)


def _sc_merge_body(p_ref, o_ref):
    lists = [[p_ref[0, t, k:k + 1, :] for k in range(8)]
             for t in range(_SHARDS)]
    while len(lists) > 1:
        lists = [_merge_keep_top(lists[2 * i], lists[2 * i + 1])
                 for i in range(len(lists) // 2)]
    for k in range(8):
        o_ref[0, k:k + 1, :] = lists[0][k]


def _sc_merge(part):
    n_sc, nsh, _, C = part.shape
    return pl.pallas_call(
        _sc_merge_body,
        grid=(n_sc,),
        in_specs=[pl.BlockSpec((1, nsh, 8, C), lambda b: (b, 0, 0, 0))],
        out_specs=pl.BlockSpec((1, 8, C), lambda b: (b, 0, 0)),
        out_shape=jax.ShapeDtypeStruct((n_sc, 8, C), jnp.float32),
    )(part)


def kernel(x):
    B, S, C = x.shape
    n_tc = B - _N_SC
    part = _sc_topk_partials(x[n_tc:])
    out_tc = _tc_topk(x[:n_tc])
    out_sc = _sc_merge(part)
    out = jnp.concatenate([out_tc, out_sc], axis=0)
    return out.reshape(B, 8 * C)


# speedup vs baseline: 3.0369x; 3.0203x over previous
import jax
import jax.numpy as jnp
from jax import lax
from jax.experimental import pallas as pl
from jax.experimental.pallas import tpu as pltpu

_BATCHER8 = [
    (0, 1), (2, 3), (4, 5), (6, 7),
    (0, 2), (1, 3), (4, 6), (5, 7),
    (1, 2), (5, 6),
    (0, 4), (1, 5), (2, 6), (3, 7),
    (2, 4), (3, 5),
    (1, 2), (3, 4), (5, 6),
]

_BITONIC8 = [
    (0, 4), (1, 5), (2, 6), (3, 7),
    (0, 2), (1, 3), (4, 6), (5, 7),
    (0, 1), (2, 3), (4, 5), (6, 7),
]


def _ce(a, i, j):
    hi = jnp.maximum(a[i], a[j])
    lo = jnp.minimum(a[i], a[j])
    a[i] = hi
    a[j] = lo


def _merge_keep_top(a, b):
    m = [jnp.maximum(a[i], b[7 - i]) for i in range(8)]
    for (i, j) in _BITONIC8:
        _ce(m, i, j)
    return m


def _sorted_group(x_ref, b, start):
    g = [x_ref[b, pl.ds(pl.multiple_of(start + 8 * k, 8), 8), :]
         for k in range(8)]
    for (i, j) in _BATCHER8:
        _ce(g, i, j)
    return g


def _body(x_ref, o_ref):
    NB, S, C = x_ref.shape
    niter = S // 128

    for b in range(NB):
        neg = jnp.full((8, C), -jnp.inf, jnp.float32)

        def step(m, carry):
            st0, st1 = carry
            base = m * 128
            st0 = _merge_keep_top(list(st0), _sorted_group(x_ref, b, base))
            st1 = _merge_keep_top(list(st1), _sorted_group(x_ref, b, base + 64))
            return (tuple(st0), tuple(st1))

        st0, st1 = lax.fori_loop(
            0, niter, step, (tuple([neg] * 8), tuple([neg] * 8)), unroll=8)
        c = _merge_keep_top(list(st0), list(st1))
        for shift in (4, 2, 1):
            rolled = [pltpu.roll(c[k], shift, axis=0) for k in range(8)]
            c = _merge_keep_top(c, rolled)
        for k in range(8):
            o_ref[b, k:k + 1, :] = c[k][0:1, :]


def kernel(x):
    B, S, C = x.shape
    NB = 4
    out = pl.pallas_call(
        _body,
        grid=(B // NB,),
        in_specs=[pl.BlockSpec((NB, S, C), lambda b: (b, 0, 0))],
        out_specs=pl.BlockSpec((NB, 8, C), lambda b: (b, 0, 0)),
        out_shape=jax.ShapeDtypeStruct((B, 8, C), x.dtype),
        compiler_params=pltpu.CompilerParams(vmem_limit_bytes=110 * 1024 * 1024),
    )(x)
    return out.reshape(B, 8 * C)


# speedup vs baseline: 3.0635x; 1.0088x over previous
import jax
import jax.numpy as jnp
from jax import lax
from jax.experimental import pallas as pl
from jax.experimental.pallas import tpu as pltpu

_BATCHER8 = [
    (0, 1), (2, 3), (4, 5), (6, 7),
    (0, 2), (1, 3), (4, 6), (5, 7),
    (1, 2), (5, 6),
    (0, 4), (1, 5), (2, 6), (3, 7),
    (2, 4), (3, 5),
    (1, 2), (3, 4), (5, 6),
]

_BITONIC8 = [
    (0, 4), (1, 5), (2, 6), (3, 7),
    (0, 2), (1, 3), (4, 6), (5, 7),
    (0, 1), (2, 3), (4, 5), (6, 7),
]


def _ce(a, i, j):
    hi = jnp.maximum(a[i], a[j])
    lo = jnp.minimum(a[i], a[j])
    a[i] = hi
    a[j] = lo


def _merge_keep_top(a, b):
    m = [jnp.maximum(a[i], b[7 - i]) for i in range(8)]
    for (i, j) in _BITONIC8:
        _ce(m, i, j)
    return m


def _sorted_group(x_ref, b, start):
    g = [x_ref[b, pl.ds(pl.multiple_of(start + 8 * k, 8), 8), :]
         for k in range(8)]
    for (i, j) in _BATCHER8:
        _ce(g, i, j)
    return g


def _body(x_ref, o_ref):
    NB, S, C = x_ref.shape
    niter = S // 128

    for b in range(NB):
        neg = jnp.full((8, C), -jnp.inf, jnp.float32)

        def step(m, carry):
            st0, st1 = carry
            base = m * 128
            st0 = _merge_keep_top(list(st0), _sorted_group(x_ref, b, base))
            st1 = _merge_keep_top(list(st1), _sorted_group(x_ref, b, base + 64))
            return (tuple(st0), tuple(st1))

        st0, st1 = lax.fori_loop(
            0, niter, step, (tuple([neg] * 8), tuple([neg] * 8)), unroll=16)
        c = _merge_keep_top(list(st0), list(st1))
        for shift in (4, 2, 1):
            rolled = [pltpu.roll(c[k], shift, axis=0) for k in range(8)]
            c = _merge_keep_top(c, rolled)
        for k in range(8):
            o_ref[b, k:k + 1, :] = c[k][0:1, :]


def kernel(x):
    B, S, C = x.shape
    NB = 4
    out = pl.pallas_call(
        _body,
        grid=(B // NB,),
        in_specs=[pl.BlockSpec((NB, S, C), lambda b: (b, 0, 0))],
        out_specs=pl.BlockSpec((NB, 8, C), lambda b: (b, 0, 0)),
        out_shape=jax.ShapeDtypeStruct((B, 8, C), x.dtype),
    )(x)
    return out.reshape(B, 8 * C)
